# baseline (speedup 1.0000x reference)
import functools
import jax
import jax.numpy as jnp
from jax import lax
from jax.experimental import pallas as pl
from jax.experimental.pallas import tpu as pltpu
from jax.experimental.pallas import tpu_sc as plsc

D_MODEL = 64
SCALE = 8.0
LANES = 16

NUM_CORES = 2
NUM_SUBCORES = 16
NUM_WORKERS = NUM_CORES * NUM_SUBCORES

N_ROWS = 4096
N_COLS = 200
ROWS_PER_WORKER = N_ROWS // NUM_WORKERS

CHUNK_A = 104
CHUNK_B = 96
SLOT_OFF = (0, CHUNK_A, 0, CHUNK_A)
SLOT_LEN = (CHUNK_A, CHUNK_B, CHUNK_A, CHUNK_B)
NBUF = 4
LOOKAHEAD = 2
CHUNKS_PER_WORKER = 2 * ROWS_PER_WORKER

_mesh = plsc.VectorSubcoreMesh(core_axis_name="c", subcore_axis_name="s")


@functools.partial(
    pl.kernel,
    out_type=jax.ShapeDtypeStruct((N_ROWS, N_COLS, D_MODEL), jnp.float32),
    mesh=_mesh,
    scratch_types=[
        pltpu.VMEM((ROWS_PER_WORKER, N_COLS), jnp.int32),
        [pltpu.VMEM((SLOT_LEN[b], D_MODEL), jnp.float32) for b in range(NBUF)],
        [pltpu.SemaphoreType.DMA] * NBUF,
        [pltpu.SemaphoreType.DMA] * NBUF,
    ],
    compiler_params=pltpu.CompilerParams(use_tc_tiling_on_sc=False),
)
def _embed(x_hbm, table_hbm, out_hbm, idx_v, rows, gsem, ssem):
    wid = lax.axis_index("s") * NUM_CORES + lax.axis_index("c")
    row_base = wid * ROWS_PER_WORKER
    pltpu.sync_copy(x_hbm.at[pl.ds(row_base, ROWS_PER_WORKER)], idx_v)

    def gather(r, b):
        src = table_hbm.at[idx_v.at[r, pl.ds(SLOT_OFF[b], SLOT_LEN[b])]]
        return pltpu.async_copy(src, rows[b], gsem[b])

    def scatter(r, b):
        dst = out_hbm.at[row_base + r, pl.ds(SLOT_OFF[b], SLOT_LEN[b])]
        return pltpu.make_async_copy(rows[b], dst, ssem[b])

    for b in range(LOOKAHEAD):
        gather(0, b)

    def outer(i, carry):
        r0 = 2 * i
        for j in range(NBUF):
            g = 4 * i + j
            r = r0 + (j // 2)
            b = j
            b2 = (j + LOOKAHEAD) % NBUF
            gl = g + LOOKAHEAD

            @pl.when(jnp.logical_and(gl >= NBUF, gl < CHUNKS_PER_WORKER))
            def _():
                scatter((gl - NBUF) // 2, b2).wait()

            @pl.when(gl < CHUNKS_PER_WORKER)
            def _():
                gather(gl // 2, b2)

            src = table_hbm.at[idx_v.at[r, pl.ds(SLOT_OFF[b], SLOT_LEN[b])]]
            pltpu.make_async_copy(src, rows[b], gsem[b]).wait()

            @plsc.parallel_loop(0, SLOT_LEN[b], step=1, unroll=8)
            def _(rr):
                for jj in range(D_MODEL // LANES):
                    sl = pl.ds(jj * LANES, LANES)
                    rows[b][rr, sl] = rows[b][rr, sl] * SCALE

            scatter(r, b).start()
        return carry

    lax.fori_loop(0, ROWS_PER_WORKER // 2, outer, 0)

    for j in range(NBUF):
        g = CHUNKS_PER_WORKER - NBUF + j
        scatter(g // 2, j).wait()


def kernel(x, table):
    return _embed(x.astype(jnp.int32), table)
